# baseline (speedup 1.0000x reference)
import jax
import jax.numpy as jnp
from jax import lax
from jax.experimental import pallas as pl
from jax.experimental.pallas import tpu as pltpu
from jax.experimental.pallas import tpu_sc as plsc

EPS = 1e-5
NC = 2
NS = 16
NT = NC * NS
EK = 128
CH0_FRAC_NUM = 7
CH0_FRAC_DEN = 10
CH0B_FRAC_NUM = 3
CH0B_FRAC_DEN = 4


def _mesh():
    return plsc.VectorSubcoreMesh(core_axis_name="c", subcore_axis_name="s")


def _sc_degree(dstp, npad, ch0, ch1):
    chf = dstp.shape[2]
    rpt = npad // NS

    def body(dst_hbm, out_hbm, dst_v, ones_v, z_v, acc_s):
        c = lax.axis_index("c")
        s = lax.axis_index("s")

        def init_ones(i, carry):
            ones_v[pl.ds(i * 16, 16)] = jnp.ones((16,), jnp.float32)
            return carry

        lax.fori_loop(0, EK // 16, init_ones, 0)

        def init_zero(i, carry):
            z_v[pl.ds(i * 16, 16)] = jnp.zeros((16,), jnp.float32)
            return carry

        lax.fori_loop(0, rpt // 16, init_zero, 0)

        pltpu.sync_copy(z_v, acc_s.at[pl.ds(s * rpt, rpt)])

        @pl.when(c == 0)
        def _():
            pltpu.sync_copy(dst_hbm.at[0, s], dst_v.at[0])
            pltpu.sync_copy(dst_hbm.at[1, s], dst_v.at[1])

        plsc.subcore_barrier()

        @pl.when(c == 0)
        def _():
            for cc, nch in ((0, ch0), (1, ch1)):
                def step(j, carry, cc=cc):
                    pltpu.sync_copy(ones_v, acc_s.at[dst_v.at[cc, j]],
                                    add=True)
                    return carry

                lax.fori_loop(0, nch, step, 0)

        plsc.subcore_barrier()
        pltpu.sync_copy(acc_s.at[pl.ds(s * rpt, rpt)],
                        out_hbm.at[c, pl.ds(s * rpt, rpt)])

    f = pl.kernel(
        body,
        out_type=jax.ShapeDtypeStruct((NC, npad), jnp.float32),
        mesh=_mesh(),
        scratch_types=[
            pltpu.VMEM((NC, chf, EK), jnp.int32),
            pltpu.VMEM((EK,), jnp.float32),
            pltpu.VMEM((rpt,), jnp.float32),
            pltpu.VMEM_SHARED((npad,), jnp.float32),
        ],
    )
    return f(dstp)


def _sc_propagate(h, srcp, dstp, npad, ch0, ch1, width):
    n = h.shape[0]
    chf = srcp.shape[2]
    rpt = npad // NS
    assert rpt % EK == 0

    def body(h_hbm, src_hbm, dst_hbm, out_hbm, src_v, dst_v, rows0_v,
             acc_s, sem0):
        c = lax.axis_index("c")
        s = lax.axis_index("s")
        nch = jnp.where(c == 0, ch0, ch1)

        def init_zero(i, carry):
            r = i // (width // 16)
            k = i % (width // 16)
            rows0_v[r, pl.ds(k * 16, 16)] = jnp.zeros((16,), jnp.float32)
            return carry

        lax.fori_loop(0, EK * (width // 16), init_zero, 0)

        pltpu.sync_copy(src_hbm.at[c, s], src_v)
        pltpu.sync_copy(dst_hbm.at[c, s], dst_v)

        def zcopy(t, carry):
            pltpu.sync_copy(rows0_v, acc_s.at[pl.ds(s * rpt + t * EK, EK)])
            return carry

        lax.fori_loop(0, rpt // EK, zcopy, 0)
        plsc.subcore_barrier()

        def step(j, carry):
            pltpu.async_copy(h_hbm.at[src_v.at[j]], rows0_v, sem0).wait()
            pltpu.sync_copy(rows0_v, acc_s.at[dst_v.at[j]], add=True)
            return carry

        lax.fori_loop(0, nch, step, 0)
        plsc.subcore_barrier()
        pltpu.sync_copy(acc_s.at[pl.ds(s * rpt, rpt)],
                        out_hbm.at[c, pl.ds(s * rpt, rpt)])

    f = pl.kernel(
        body,
        out_type=jax.ShapeDtypeStruct((NC, npad, width), jnp.float32),
        mesh=_mesh(),
        scratch_types=[
            pltpu.VMEM((chf, EK), jnp.int32),
            pltpu.VMEM((chf, EK), jnp.int32),
            pltpu.VMEM((EK, width), jnp.float32),
            pltpu.VMEM_SHARED((npad, width), jnp.float32),
            pltpu.SemaphoreType.DMA,
        ],
    )
    return f(h, srcp, dstp)


def _tc_first(x, W, degT):
    n, d = x.shape
    h = W.shape[1]
    R = 2000

    def body(x_ref, w_ref, deg_ref, h_ref, dinv_ref):
        dg = deg_ref[...]
        dinv = lax.rsqrt(dg[:, 0] + dg[:, 1] + 1.0)[:, None]
        hh = jnp.dot(x_ref[...], w_ref[...], preferred_element_type=jnp.float32)
        h_ref[...] = hh * dinv
        dinv_ref[...] = dinv

    return pl.pallas_call(
        body,
        grid=(n // R,),
        in_specs=[
            pl.BlockSpec((R, d), lambda i: (i, 0)),
            pl.BlockSpec((d, h), lambda i: (0, 0)),
            pl.BlockSpec((R, 2), lambda i: (i, 0)),
        ],
        out_specs=[
            pl.BlockSpec((R, h), lambda i: (i, 0)),
            pl.BlockSpec((R, 1), lambda i: (i, 0)),
        ],
        out_shape=[
            jax.ShapeDtypeStruct((n, h), jnp.float32),
            jax.ShapeDtypeStruct((n, 1), jnp.float32),
        ],
    )(x, W, degT)


def _tc_mid(p, hprev, dinv, b, g, be, m, v, W):
    n, hin = hprev.shape
    hout = W.shape[1]
    R = 2000

    def body(p_ref, hp_ref, dinv_ref, b_ref, g_ref, be_ref, m_ref, v_ref,
             w_ref, o_ref):
        dv = dinv_ref[...]
        z = dv * (p_ref[0] + p_ref[1] + hp_ref[...]) + b_ref[...][None, :]
        sc = (lax.rsqrt(v_ref[...] + EPS) * g_ref[...])[None, :]
        z = (z - m_ref[...][None, :]) * sc + be_ref[...][None, :]
        z = jnp.maximum(z, 0.0)
        o_ref[...] = jnp.dot(z, w_ref[...],
                             preferred_element_type=jnp.float32) * dv

    return pl.pallas_call(
        body,
        grid=(n // R,),
        in_specs=[
            pl.BlockSpec((2, R, hin), lambda i: (0, i, 0)),
            pl.BlockSpec((R, hin), lambda i: (i, 0)),
            pl.BlockSpec((R, 1), lambda i: (i, 0)),
            pl.BlockSpec((hin,), lambda i: (0,)),
            pl.BlockSpec((hin,), lambda i: (0,)),
            pl.BlockSpec((hin,), lambda i: (0,)),
            pl.BlockSpec((hin,), lambda i: (0,)),
            pl.BlockSpec((hin,), lambda i: (0,)),
            pl.BlockSpec((hin, hout), lambda i: (0, 0)),
        ],
        out_specs=pl.BlockSpec((R, hout), lambda i: (i, 0)),
        out_shape=jax.ShapeDtypeStruct((n, hout), jnp.float32),
    )(p, hprev, dinv, b, g, be, m, v, W)


def _tc_act(p, hprev, dinv, b, g, be, m, v):
    n, hin = hprev.shape
    R = 2000

    def body(p_ref, hp_ref, dinv_ref, b_ref, g_ref, be_ref, m_ref, v_ref,
             o_ref):
        dv = dinv_ref[...]
        z = dv * (p_ref[0] + p_ref[1] + hp_ref[...]) + b_ref[...][None, :]
        sc = (lax.rsqrt(v_ref[...] + EPS) * g_ref[...])[None, :]
        z = (z - m_ref[...][None, :]) * sc + be_ref[...][None, :]
        o_ref[...] = jnp.maximum(z, 0.0) * dv

    return pl.pallas_call(
        body,
        grid=(n // R,),
        in_specs=[
            pl.BlockSpec((2, R, hin), lambda i: (0, i, 0)),
            pl.BlockSpec((R, hin), lambda i: (i, 0)),
            pl.BlockSpec((R, 1), lambda i: (i, 0)),
            pl.BlockSpec((hin,), lambda i: (0,)),
            pl.BlockSpec((hin,), lambda i: (0,)),
            pl.BlockSpec((hin,), lambda i: (0,)),
            pl.BlockSpec((hin,), lambda i: (0,)),
            pl.BlockSpec((hin,), lambda i: (0,)),
        ],
        out_specs=pl.BlockSpec((R, hin), lambda i: (i, 0)),
        out_shape=jax.ShapeDtypeStruct((n, hin), jnp.float32),
    )(p, hprev, dinv, b, g, be, m, v)


def _tc_final(p, hprev, dinv, W, b):
    n, hin = hprev.shape
    c = W.shape[1]
    R = 2000

    def body(p_ref, hp_ref, dinv_ref, w_ref, b_ref, o_ref):
        dv = dinv_ref[...]
        q = dv * (p_ref[0] + p_ref[1] + hp_ref[...])
        o_ref[...] = jnp.dot(q, w_ref[...],
                             preferred_element_type=jnp.float32) + b_ref[...][None, :]

    return pl.pallas_call(
        body,
        grid=(n // R,),
        in_specs=[
            pl.BlockSpec((2, R, hin), lambda i: (0, i, 0)),
            pl.BlockSpec((R, hin), lambda i: (i, 0)),
            pl.BlockSpec((R, 1), lambda i: (i, 0)),
            pl.BlockSpec((hin, c), lambda i: (0, 0)),
            pl.BlockSpec((c,), lambda i: (0,)),
        ],
        out_specs=pl.BlockSpec((R, c), lambda i: (i, 0)),
        out_shape=jax.ShapeDtypeStruct((n, c), jnp.float32),
    )(p, hprev, dinv, W, b)


def kernel(x, edge_index, W1, b1, g1, be1, m1, v1, W2, b2, g2, be2, m2, v2,
           W3, b3):
    n, d = x.shape
    e = edge_index.shape[1]
    npad = -(-n // (NS * 16)) * (NS * 16)

    src = edge_index[0]
    dst = edge_index[1]

    def layout(num, den):
        ch_total = 2 * (-(-e // (2 * NS * EK)))
        ch0 = num * ch_total // den
        ch1 = ch_total - ch0
        chf = max(ch0, ch1)
        cap0 = NS * ch0 * EK
        cap1 = NS * ch1 * EK
        srcf = jnp.concatenate(
            [src, jnp.zeros((cap0 + cap1 - e,), jnp.int32)])
        dstf = jnp.concatenate(
            [dst, jnp.full((cap0 + cap1 - e,), n, jnp.int32)])
        s0 = jnp.pad(srcf[:cap0].reshape(NS, ch0, EK),
                     ((0, 0), (0, chf - ch0), (0, 0)))
        s1 = jnp.pad(srcf[cap0:].reshape(NS, ch1, EK),
                     ((0, 0), (0, chf - ch1), (0, 0)))
        d0 = jnp.pad(dstf[:cap0].reshape(NS, ch0, EK),
                     ((0, 0), (0, chf - ch0), (0, 0)), constant_values=n)
        d1 = jnp.pad(dstf[cap0:].reshape(NS, ch1, EK),
                     ((0, 0), (0, chf - ch1), (0, 0)), constant_values=n)
        return jnp.stack([s0, s1]), jnp.stack([d0, d1]), ch0, ch1

    srcpb, dstpb, ch0b, ch1b = layout(CH0B_FRAC_NUM, CH0B_FRAC_DEN)

    deg = _sc_degree(dstpb, npad, ch0b, ch1b)
    degT = deg.T

    h1, dinv = _tc_first(x, W1, degT)
    p1 = _sc_propagate(h1, srcpb, dstpb, npad, ch0b, ch1b, 128)
    h2 = _tc_mid(p1, h1, dinv, b1, g1, be1, m1, v1, W2)
    p2 = _sc_propagate(h2, srcpb, dstpb, npad, ch0b, ch1b, 128)
    z2 = _tc_act(p2, h2, dinv, b2, g2, be2, m2, v2)
    p3 = _sc_propagate(z2, srcpb, dstpb, npad, ch0b, ch1b, 128)
    return _tc_final(p3, z2, dinv, W3, b3)


# speedup vs baseline: 1.0380x; 1.0380x over previous
import jax
import jax.numpy as jnp
from jax import lax
from jax.experimental import pallas as pl
from jax.experimental.pallas import tpu as pltpu
from jax.experimental.pallas import tpu_sc as plsc

EPS = 1e-5
NC = 2
NS = 16
NT = NC * NS
EK = 128
CH0_FRAC_NUM = 7
CH0_FRAC_DEN = 10
CH0B_FRAC_NUM = 3
CH0B_FRAC_DEN = 4


def _mesh():
    return plsc.VectorSubcoreMesh(core_axis_name="c", subcore_axis_name="s")


def _sc_degree(dstp, npad, ch0, ch1):
    chf = dstp.shape[2]
    rpt = npad // NS

    def body(dst_hbm, out_hbm, dst_v, ones_v, z_v, acc_s):
        c = lax.axis_index("c")
        s = lax.axis_index("s")

        def init_ones(i, carry):
            ones_v[pl.ds(i * 16, 16)] = jnp.ones((16,), jnp.float32)
            return carry

        lax.fori_loop(0, EK // 16, init_ones, 0)

        def init_zero(i, carry):
            z_v[pl.ds(i * 16, 16)] = jnp.zeros((16,), jnp.float32)
            return carry

        lax.fori_loop(0, rpt // 16, init_zero, 0)

        pltpu.sync_copy(z_v, acc_s.at[pl.ds(s * rpt, rpt)])

        @pl.when(c == 0)
        def _():
            pltpu.sync_copy(dst_hbm.at[0, s], dst_v.at[0])
            pltpu.sync_copy(dst_hbm.at[1, s], dst_v.at[1])

        plsc.subcore_barrier()

        @pl.when(c == 0)
        def _():
            for cc, nch in ((0, ch0), (1, ch1)):
                def step(j, carry, cc=cc):
                    pltpu.sync_copy(ones_v, acc_s.at[dst_v.at[cc, j]],
                                    add=True)
                    return carry

                lax.fori_loop(0, nch, step, 0)

        plsc.subcore_barrier()
        pltpu.sync_copy(acc_s.at[pl.ds(s * rpt, rpt)],
                        out_hbm.at[c, pl.ds(s * rpt, rpt)])

    f = pl.kernel(
        body,
        out_type=jax.ShapeDtypeStruct((NC, npad), jnp.float32),
        mesh=_mesh(),
        scratch_types=[
            pltpu.VMEM((NC, chf, EK), jnp.int32),
            pltpu.VMEM((EK,), jnp.float32),
            pltpu.VMEM((rpt,), jnp.float32),
            pltpu.VMEM_SHARED((npad,), jnp.float32),
        ],
    )
    return f(dstp)


def _sc_propagate(h, srcp, dstp, npad, ch0, ch1, width):
    n = h.shape[0]
    chf = srcp.shape[2]
    rpt = npad // NS
    assert rpt % EK == 0

    def body(h_hbm, src_hbm, dst_hbm, out_hbm, src_v, dst_v, rows0_v,
             acc_s, sem0):
        c = lax.axis_index("c")
        s = lax.axis_index("s")
        nch = jnp.where(c == 0, ch0, ch1)

        def init_zero(i, carry):
            r = i // (width // 16)
            k = i % (width // 16)
            rows0_v[r, pl.ds(k * 16, 16)] = jnp.zeros((16,), jnp.float32)
            return carry

        lax.fori_loop(0, EK * (width // 16), init_zero, 0)

        pltpu.sync_copy(src_hbm.at[c, s], src_v)
        pltpu.sync_copy(dst_hbm.at[c, s], dst_v)

        def zcopy(t, carry):
            pltpu.sync_copy(rows0_v, acc_s.at[pl.ds(s * rpt + t * EK, EK)])
            return carry

        lax.fori_loop(0, rpt // EK, zcopy, 0)
        plsc.subcore_barrier()

        def step(j, carry):
            pltpu.async_copy(h_hbm.at[src_v.at[j]], rows0_v, sem0).wait()
            pltpu.sync_copy(rows0_v, acc_s.at[dst_v.at[j]], add=True)
            return carry

        lax.fori_loop(0, nch, step, 0)
        plsc.subcore_barrier()
        pltpu.sync_copy(acc_s.at[pl.ds(s * rpt, rpt)],
                        out_hbm.at[c, pl.ds(s * rpt, rpt)])

    f = pl.kernel(
        body,
        out_type=jax.ShapeDtypeStruct((NC, npad, width), jnp.float32),
        mesh=_mesh(),
        scratch_types=[
            pltpu.VMEM((chf, EK), jnp.int32),
            pltpu.VMEM((chf, EK), jnp.int32),
            pltpu.VMEM((EK, width), jnp.float32),
            pltpu.VMEM_SHARED((npad, width), jnp.float32),
            pltpu.SemaphoreType.DMA,
        ],
    )
    return f(h, srcp, dstp)


def _tc_first(x, W, degT):
    n, d = x.shape
    h = W.shape[1]
    R = 2000

    def body(x_ref, w_ref, deg_ref, h_ref, dinv_ref):
        dg = deg_ref[...]
        dinv = lax.rsqrt(dg[:, 0] + dg[:, 1] + 1.0)[:, None]
        hh = jnp.dot(x_ref[...], w_ref[...], preferred_element_type=jnp.float32)
        h_ref[...] = hh * dinv
        dinv_ref[...] = dinv

    return pl.pallas_call(
        body,
        grid=(n // R,),
        in_specs=[
            pl.BlockSpec((R, d), lambda i: (i, 0)),
            pl.BlockSpec((d, h), lambda i: (0, 0)),
            pl.BlockSpec((R, 2), lambda i: (i, 0)),
        ],
        out_specs=[
            pl.BlockSpec((R, h), lambda i: (i, 0)),
            pl.BlockSpec((R, 1), lambda i: (i, 0)),
        ],
        out_shape=[
            jax.ShapeDtypeStruct((n, h), jnp.float32),
            jax.ShapeDtypeStruct((n, 1), jnp.float32),
        ],
    )(x, W, degT)


def _tc_mid(p, hprev, dinv, b, g, be, m, v, W):
    n, hin = hprev.shape
    hout = W.shape[1]
    R = 2000

    def body(p_ref, hp_ref, dinv_ref, b_ref, g_ref, be_ref, m_ref, v_ref,
             w_ref, o_ref):
        dv = dinv_ref[...]
        z = dv * (p_ref[0] + p_ref[1] + hp_ref[...]) + b_ref[...][None, :]
        sc = (lax.rsqrt(v_ref[...] + EPS) * g_ref[...])[None, :]
        z = (z - m_ref[...][None, :]) * sc + be_ref[...][None, :]
        z = jnp.maximum(z, 0.0)
        o_ref[...] = jnp.dot(z, w_ref[...],
                             preferred_element_type=jnp.float32) * dv

    return pl.pallas_call(
        body,
        grid=(n // R,),
        in_specs=[
            pl.BlockSpec((2, R, hin), lambda i: (0, i, 0)),
            pl.BlockSpec((R, hin), lambda i: (i, 0)),
            pl.BlockSpec((R, 1), lambda i: (i, 0)),
            pl.BlockSpec((hin,), lambda i: (0,)),
            pl.BlockSpec((hin,), lambda i: (0,)),
            pl.BlockSpec((hin,), lambda i: (0,)),
            pl.BlockSpec((hin,), lambda i: (0,)),
            pl.BlockSpec((hin,), lambda i: (0,)),
            pl.BlockSpec((hin, hout), lambda i: (0, 0)),
        ],
        out_specs=pl.BlockSpec((R, hout), lambda i: (i, 0)),
        out_shape=jax.ShapeDtypeStruct((n, hout), jnp.float32),
    )(p, hprev, dinv, b, g, be, m, v, W)


def _tc_act(p, hprev, dinv, b, g, be, m, v):
    n, hin = hprev.shape
    R = 2000

    def body(p_ref, hp_ref, dinv_ref, b_ref, g_ref, be_ref, m_ref, v_ref,
             o_ref):
        dv = dinv_ref[...]
        z = dv * (p_ref[0] + p_ref[1] + hp_ref[...]) + b_ref[...][None, :]
        sc = (lax.rsqrt(v_ref[...] + EPS) * g_ref[...])[None, :]
        z = (z - m_ref[...][None, :]) * sc + be_ref[...][None, :]
        o_ref[...] = jnp.maximum(z, 0.0) * dv

    return pl.pallas_call(
        body,
        grid=(n // R,),
        in_specs=[
            pl.BlockSpec((2, R, hin), lambda i: (0, i, 0)),
            pl.BlockSpec((R, hin), lambda i: (i, 0)),
            pl.BlockSpec((R, 1), lambda i: (i, 0)),
            pl.BlockSpec((hin,), lambda i: (0,)),
            pl.BlockSpec((hin,), lambda i: (0,)),
            pl.BlockSpec((hin,), lambda i: (0,)),
            pl.BlockSpec((hin,), lambda i: (0,)),
            pl.BlockSpec((hin,), lambda i: (0,)),
        ],
        out_specs=pl.BlockSpec((R, hin), lambda i: (i, 0)),
        out_shape=jax.ShapeDtypeStruct((n, hin), jnp.float32),
    )(p, hprev, dinv, b, g, be, m, v)


def _tc_final(p, hprev, dinv, W, b):
    n, hin = hprev.shape
    c = W.shape[1]
    R = 2000

    def body(p_ref, hp_ref, dinv_ref, w_ref, b_ref, o_ref):
        dv = dinv_ref[...]
        q = dv * (p_ref[0] + p_ref[1] + hp_ref[...])
        o_ref[...] = jnp.dot(q, w_ref[...],
                             preferred_element_type=jnp.float32) + b_ref[...][None, :]

    return pl.pallas_call(
        body,
        grid=(n // R,),
        in_specs=[
            pl.BlockSpec((2, R, hin), lambda i: (0, i, 0)),
            pl.BlockSpec((R, hin), lambda i: (i, 0)),
            pl.BlockSpec((R, 1), lambda i: (i, 0)),
            pl.BlockSpec((hin, c), lambda i: (0, 0)),
            pl.BlockSpec((c,), lambda i: (0,)),
        ],
        out_specs=pl.BlockSpec((R, c), lambda i: (i, 0)),
        out_shape=jax.ShapeDtypeStruct((n, c), jnp.float32),
    )(p, hprev, dinv, W, b)


def kernel(x, edge_index, W1, b1, g1, be1, m1, v1, W2, b2, g2, be2, m2, v2,
           W3, b3):
    n, d = x.shape
    e = edge_index.shape[1]
    npad = -(-n // (NS * 16)) * (NS * 16)

    src = edge_index[0]
    dst = edge_index[1]

    def layout(num, den):
        ch_total = 2 * (-(-e // (2 * NS * EK)))
        ch0 = num * ch_total // den
        ch1 = ch_total - ch0
        chf = max(ch0, ch1)
        cap0 = NS * ch0 * EK
        cap1 = NS * ch1 * EK
        srcf = jnp.concatenate(
            [src, jnp.zeros((cap0 + cap1 - e,), jnp.int32)])
        dstf = jnp.concatenate(
            [dst, jnp.full((cap0 + cap1 - e,), n, jnp.int32)])
        s0 = jnp.pad(srcf[:cap0].reshape(NS, ch0, EK),
                     ((0, 0), (0, chf - ch0), (0, 0)))
        s1 = jnp.pad(srcf[cap0:].reshape(NS, ch1, EK),
                     ((0, 0), (0, chf - ch1), (0, 0)))
        d0 = jnp.pad(dstf[:cap0].reshape(NS, ch0, EK),
                     ((0, 0), (0, chf - ch0), (0, 0)), constant_values=n)
        d1 = jnp.pad(dstf[cap0:].reshape(NS, ch1, EK),
                     ((0, 0), (0, chf - ch1), (0, 0)), constant_values=n)
        return jnp.stack([s0, s1]), jnp.stack([d0, d1]), ch0, ch1

    srcp, dstp, ch0, ch1 = layout(CH0_FRAC_NUM, CH0_FRAC_DEN)
    srcpb, dstpb, ch0b, ch1b = layout(CH0B_FRAC_NUM, CH0B_FRAC_DEN)

    deg = _sc_degree(dstp, npad, ch0, ch1)
    degT = deg.T

    h1, dinv = _tc_first(x, W1, degT)
    p1 = _sc_propagate(h1, srcpb, dstpb, npad, ch0b, ch1b, 128)
    h2 = _tc_mid(p1, h1, dinv, b1, g1, be1, m1, v1, W2)
    p2 = _sc_propagate(h2, srcpb, dstpb, npad, ch0b, ch1b, 128)
    z2 = _tc_act(p2, h2, dinv, b2, g2, be2, m2, v2)
    p3 = _sc_propagate(z2, srcp, dstp, npad, ch0, ch1, 128)
    return _tc_final(p3, z2, dinv, W3, b3)
